# baseline (speedup 1.0000x reference)
import functools

import jax
import jax.numpy as jnp
from jax import lax
from jax.experimental import pallas as pl
from jax.experimental.pallas import tpu as pltpu
from jax.experimental.pallas import tpu_sc as plsc


def _make_sc_copy(seq_len, d_model, dtype):
    info = plsc.get_sparse_core_info()
    nc, ns = info.num_cores, info.num_subcores
    nw = nc * ns
    rows_per = seq_len // nw
    mesh = plsc.VectorSubcoreMesh(core_axis_name="c", subcore_axis_name="s")

    @functools.partial(
        pl.kernel,
        mesh=mesh,
        out_type=jax.ShapeDtypeStruct((seq_len, d_model), dtype),
    )
    def sc_copy(table_hbm, out_hbm):
        wid = lax.axis_index("c") * ns + lax.axis_index("s")
        base = wid * rows_per
        pltpu.sync_copy(
            table_hbm.at[pl.ds(base, rows_per)],
            out_hbm.at[pl.ds(base, rows_per)],
        )

    return sc_copy


def kernel(x, pos_table):
    seq_len = x.shape[1]
    d_model = pos_table.shape[1]
    table = pos_table[:seq_len]
    out = _make_sc_copy(seq_len, d_model, pos_table.dtype)(table)
    return out[None]


# speedup vs baseline: 23.1455x; 23.1455x over previous
import functools

import jax
import jax.numpy as jnp
from jax import lax
from jax.experimental import pallas as pl
from jax.experimental.pallas import tpu as pltpu
from jax.experimental.pallas import tpu_sc as plsc

_CHUNK_ROWS = 32


def _make_sc_copy(seq_len, d_model, dtype):
    info = plsc.get_sparse_core_info()
    nc, ns = info.num_cores, info.num_subcores
    nw = nc * ns
    rows_per = seq_len // nw
    nchunks = rows_per // _CHUNK_ROWS
    mesh = plsc.VectorSubcoreMesh(core_axis_name="c", subcore_axis_name="s")

    @functools.partial(
        pl.kernel,
        mesh=mesh,
        out_type=jax.ShapeDtypeStruct((seq_len, d_model), dtype),
        scratch_types=[
            pltpu.VMEM((_CHUNK_ROWS, d_model), dtype),
            pltpu.VMEM((_CHUNK_ROWS, d_model), dtype),
            pltpu.SemaphoreType.DMA,
            pltpu.SemaphoreType.DMA,
            pltpu.SemaphoreType.DMA,
            pltpu.SemaphoreType.DMA,
        ],
    )
    def sc_copy(table_hbm, out_hbm, buf0, buf1, g0, g1, s0, s1):
        wid = lax.axis_index("c") * ns + lax.axis_index("s")
        base = wid * rows_per
        bufs = (buf0, buf1)
        gsem = (g0, g1)
        ssem = (s0, s1)
        scat = [None, None]
        for i in range(nchunks):
            b = i & 1
            lo = base + i * _CHUNK_ROWS
            if scat[b] is not None:
                scat[b].wait()
            gath = pltpu.async_copy(
                table_hbm.at[pl.ds(lo, _CHUNK_ROWS)], bufs[b], gsem[b]
            )
            gath.wait()
            scat[b] = pltpu.async_copy(
                bufs[b], out_hbm.at[pl.ds(lo, _CHUNK_ROWS)], ssem[b]
            )
        for b in (0, 1):
            if scat[b] is not None:
                scat[b].wait()

    return sc_copy


def kernel(x, pos_table):
    seq_len = x.shape[1]
    d_model = pos_table.shape[1]
    table = pos_table[:seq_len]
    out = _make_sc_copy(seq_len, d_model, pos_table.dtype)(table)
    return out[None]
